# Initial kernel scaffold; baseline (speedup 1.0000x reference)
import jax
import jax.numpy as jnp
from jax.experimental import pallas as pl


def kernel(x, tables, W, b):
    raise NotImplementedError("write your pallas kernel here")


# speedup vs baseline: 6.9467x; 6.9467x over previous
import functools

import jax
import jax.numpy as jnp
from jax.experimental import pallas as pl
from jax.experimental.pallas import tpu as pltpu

B, D, EMB = 16384, 100, 64
N_CAT, VOCAB = 26, 6
BB = 256


def _fe_kernel(x_ref, tab_ref, w_ref, b_ref, out_ref):
    xb = x_ref[...]
    idx = jnp.clip(xb[:, :N_CAT].astype(jnp.int32), 0, VOCAB - 1)
    idx3 = idx[:, :, None]
    acc = jnp.zeros((xb.shape[0], N_CAT, EMB), jnp.float32)
    for v in range(VOCAB):
        acc = jnp.where(idx3 == v, tab_ref[:, v, :][None, :, :], acc)
    dense = (
        xb[:, N_CAT:, None] * w_ref[0][None, None, :]
        + b_ref[0][None, None, :]
    )
    out_ref[...] = jnp.concatenate([acc, dense], axis=1)


@jax.jit
def kernel(x, tables, W, b):
    b2 = b.reshape(1, EMB)
    grid = (B // BB,)
    out = pl.pallas_call(
        _fe_kernel,
        grid=grid,
        in_specs=[
            pl.BlockSpec((BB, D), lambda i: (i, 0)),
            pl.BlockSpec((N_CAT, VOCAB, EMB), lambda i: (0, 0, 0)),
            pl.BlockSpec((1, EMB), lambda i: (0, 0)),
            pl.BlockSpec((1, EMB), lambda i: (0, 0)),
        ],
        out_specs=pl.BlockSpec((BB, D, EMB), lambda i: (i, 0, 0)),
        out_shape=jax.ShapeDtypeStruct((B, D, EMB), jnp.float32),
        compiler_params=pltpu.CompilerParams(
            dimension_semantics=("arbitrary",),
        ),
    )(x, tables, W, b2)
    return out


# speedup vs baseline: 8.8834x; 1.2788x over previous
import jax
import jax.numpy as jnp
from jax.experimental import pallas as pl
from jax.experimental.pallas import tpu as pltpu

B, D, EMB = 16384, 100, 64
N_CAT, VOCAB = 26, 6
N_DEN = D - N_CAT
CATW = N_CAT * EMB
DENW = N_DEN * EMB
BB = 256


def _fe_kernel(xc_ref, xd_ref, r64_ref, rdw_ref, trow_ref, bt_ref, out_ref):
    idx_f = jnp.clip(xc_ref[...].astype(jnp.int32), 0, VOCAB - 1).astype(
        jnp.float32
    )
    idx_rep = jnp.dot(
        idx_f, r64_ref[...], preferred_element_type=jnp.float32
    ).astype(jnp.int32)
    acc = jnp.broadcast_to(trow_ref[0:1, :], idx_rep.shape)
    for v in range(1, VOCAB):
        acc = jnp.where(idx_rep == v, trow_ref[v : v + 1, :], acc)
    out_ref[:, :CATW] = acc
    den = (
        jnp.dot(
            xd_ref[...],
            rdw_ref[...],
            preferred_element_type=jnp.float32,
            precision=jax.lax.Precision.HIGHEST,
        )
        + bt_ref[...]
    )
    out_ref[:, CATW:] = den


@jax.jit
def kernel(x, tables, W, b):
    xc = x[:, :N_CAT]
    xd = x[:, N_CAT:]
    eye26 = jnp.eye(N_CAT, dtype=jnp.float32)
    r64 = jnp.repeat(eye26, EMB, axis=1)
    rdw = jnp.kron(jnp.eye(N_DEN, dtype=jnp.float32), W)
    trow = tables.transpose(1, 0, 2).reshape(VOCAB, CATW)
    bt = jnp.tile(b, N_DEN).reshape(1, DENW)
    grid = (B // BB,)
    out2d = pl.pallas_call(
        _fe_kernel,
        grid=grid,
        in_specs=[
            pl.BlockSpec((BB, N_CAT), lambda i: (i, 0)),
            pl.BlockSpec((BB, N_DEN), lambda i: (i, 0)),
            pl.BlockSpec((N_CAT, CATW), lambda i: (0, 0)),
            pl.BlockSpec((N_DEN, DENW), lambda i: (0, 0)),
            pl.BlockSpec((VOCAB, CATW), lambda i: (0, 0)),
            pl.BlockSpec((1, DENW), lambda i: (0, 0)),
        ],
        out_specs=pl.BlockSpec((BB, D * EMB), lambda i: (i, 0)),
        out_shape=jax.ShapeDtypeStruct((B, D * EMB), jnp.float32),
        compiler_params=pltpu.CompilerParams(
            dimension_semantics=("arbitrary",),
        ),
    )(xc, xd, r64, rdw, trow, bt)
    return out2d.reshape(B, D, EMB)


# speedup vs baseline: 8.9389x; 1.0062x over previous
import jax
import jax.numpy as jnp
from jax.experimental import pallas as pl
from jax.experimental.pallas import tpu as pltpu

B, D, EMB = 16384, 100, 64
N_CAT, VOCAB = 26, 6
N_DEN = D - N_CAT
CATW = N_CAT * EMB
DENW = N_DEN * EMB
BB = 512


def _fe_kernel(xc_ref, xd_ref, r64_ref, rdw_ref, trow_ref, bt_ref, out_ref):
    idx_f = jnp.clip(xc_ref[...].astype(jnp.int32), 0, VOCAB - 1).astype(
        jnp.float32
    )
    idx_rep = jnp.dot(
        idx_f, r64_ref[...], preferred_element_type=jnp.float32
    ).astype(jnp.int32)
    acc = jnp.broadcast_to(trow_ref[0:1, :], idx_rep.shape)
    for v in range(1, VOCAB):
        acc = jnp.where(idx_rep == v, trow_ref[v : v + 1, :], acc)
    out_ref[:, :CATW] = acc
    den = (
        jnp.dot(
            xd_ref[...],
            rdw_ref[...],
            preferred_element_type=jnp.float32,
            precision=jax.lax.Precision.HIGHEST,
        )
        + bt_ref[...]
    )
    out_ref[:, CATW:] = den


@jax.jit
def kernel(x, tables, W, b):
    xc = x[:, :N_CAT]
    xd = x[:, N_CAT:]
    eye26 = jnp.eye(N_CAT, dtype=jnp.float32)
    r64 = jnp.repeat(eye26, EMB, axis=1)
    rdw = jnp.kron(jnp.eye(N_DEN, dtype=jnp.float32), W)
    trow = tables.transpose(1, 0, 2).reshape(VOCAB, CATW)
    bt = jnp.tile(b, N_DEN).reshape(1, DENW)
    grid = (B // BB,)
    out2d = pl.pallas_call(
        _fe_kernel,
        grid=grid,
        in_specs=[
            pl.BlockSpec((BB, N_CAT), lambda i: (i, 0)),
            pl.BlockSpec((BB, N_DEN), lambda i: (i, 0)),
            pl.BlockSpec((N_CAT, CATW), lambda i: (0, 0)),
            pl.BlockSpec((N_DEN, DENW), lambda i: (0, 0)),
            pl.BlockSpec((VOCAB, CATW), lambda i: (0, 0)),
            pl.BlockSpec((1, DENW), lambda i: (0, 0)),
        ],
        out_specs=pl.BlockSpec((BB, D * EMB), lambda i: (i, 0)),
        out_shape=jax.ShapeDtypeStruct((B, D * EMB), jnp.float32),
        compiler_params=pltpu.CompilerParams(
            dimension_semantics=("arbitrary",),
        ),
    )(xc, xd, r64, rdw, trow, bt)
    return out2d.reshape(B, D, EMB)
